# baseline (speedup 1.0000x reference)
import jax
import jax.numpy as jnp
from jax import lax
from jax.experimental import pallas as pl
from jax.experimental.pallas import tpu as pltpu
from jax.experimental.pallas import tpu_sc as plsc

NUM_CORES = 2
NUM_SUBCORES = 16
NUM_WORKERS = NUM_CORES * NUM_SUBCORES

V = 1_000_000
D = 32
T = 200
B = 16384

VBLK = 128
NFULL = V // VBLK
TAIL = V - NFULL * VBLK
NUNITS = (T // 8) * (B // 128)
NU = NUNITS // NUM_WORKERS


def _wid():
    return lax.axis_index("s") * NUM_CORES + lax.axis_index("c")


def _wfmt_body(wt_hbm, wpad_hbm, band_v, tr_v, band_t, tr_t,
               sem_i0, sem_i1, sem_o0, sem_o1):
    wid = _wid()
    nblk = (NFULL - wid + NUM_WORKERS - 1) // NUM_WORKERS
    iota16 = lax.iota(jnp.int32, 16)
    sem_i = (sem_i0, sem_i1)
    sem_o = (sem_o0, sem_o1)

    def start_in(vb, bb):
        pltpu.make_async_copy(
            wt_hbm.at[:, pl.ds(vb * VBLK, VBLK)], band_v.at[bb], sem_i[bb]
        ).start()

    def wait_in(bb):
        pltpu.make_async_copy(
            wt_hbm.at[:, pl.ds(0, VBLK)], band_v.at[bb], sem_i[bb]
        ).wait()

    def start_out(vb, bb):
        pltpu.make_async_copy(
            tr_v.at[bb], wpad_hbm.at[pl.ds(vb * VBLK, VBLK)], sem_o[bb]
        ).start()

    def wait_out(bb):
        pltpu.make_async_copy(
            tr_v.at[bb], wpad_hbm.at[pl.ds(0, VBLK)], sem_o[bb]
        ).wait()

    def transpose_block(bb):
        def dbody(d, carry):
            col = jnp.full((16,), d, jnp.int32)
            for j0 in range(8):
                vec = band_v[bb, d, pl.ds(j0 * 16, 16)]
                plsc.store_scatter(tr_v.at[bb], [iota16 + j0 * 16, col], vec)
            return carry
        lax.fori_loop(0, D, dbody, 0, unroll=False)

    start_in(wid, 0)
    start_in(wid + NUM_WORKERS, 1)
    for bb in range(2):
        wait_in(bb)
        transpose_block(bb)
        start_out(wid + bb * NUM_WORKERS, bb)
        start_in(wid + (bb + 2) * NUM_WORKERS, bb)

    def group(g, carry):
        for bb in range(2):
            k = 2 + g * 2 + bb
            vb = wid + k * NUM_WORKERS

            @pl.when(k < nblk)
            def _():
                wait_in(bb)
                wait_out(bb)
                transpose_block(bb)
                start_out(vb, bb)

            @pl.when(k + 2 < nblk)
            def _():
                pltpu.make_async_copy(
                    wt_hbm.at[:, pl.ds((vb + 2 * NUM_WORKERS) * VBLK, VBLK)],
                    band_v.at[bb], sem_i[bb],
                ).start()
        return carry

    ngroups = (245 - 2 + 1) // 2
    lax.fori_loop(0, ngroups, group, 0, unroll=False)
    for bb in range(2):
        wait_out(bb)

    @pl.when(wid == NUM_WORKERS - 1)
    def _():
        pltpu.sync_copy(wt_hbm.at[:, pl.ds(NFULL * VBLK, TAIL)], band_t)

        def dbody(d, carry):
            col = jnp.full((16,), d, jnp.int32)
            for j0 in range(TAIL // 16):
                vec = band_t[d, pl.ds(j0 * 16, 16)]
                plsc.store_scatter(tr_t, [iota16 + j0 * 16, col], vec)
            return carry
        lax.fori_loop(0, D, dbody, 0, unroll=False)
        pltpu.sync_copy(tr_t, wpad_hbm.at[pl.ds(NFULL * VBLK, TAIL)])


def _gather_body(ids_hbm, wpad_hbm, out_hbm,
                 idx_v, rows_v, tile_v,
                 sem_x0, sem_x1, sem_g0, sem_g1,
                 sem_o0, sem_o1, sem_o2, sem_o3):
    wid = _wid()
    iota16 = lax.iota(jnp.int32, 16)
    sem_x = (sem_x0, sem_x1)
    sem_g = (sem_g0, sem_g1)
    sem_o = (sem_o0, sem_o1)
    del sem_o2, sem_o3
    u0 = wid * NU

    def start_idx(u, up):
        tt = u // 128
        bc = lax.rem(u, 128)
        pltpu.make_async_copy(
            ids_hbm.at[pl.ds(tt * 8, 8), pl.ds(bc * 128, 128)],
            idx_v.at[up], sem_x[up],
        ).start()

    def wait_idx(up):
        pltpu.make_async_copy(
            ids_hbm.at[pl.ds(0, 8), pl.ds(0, 128)], idx_v.at[up], sem_x[up]
        ).wait()

    def start_gather(up, r, gb):
        pltpu.make_async_copy(
            wpad_hbm.at[idx_v.at[up, r]], rows_v.at[gb], sem_g[gb]
        ).start()

    def wait_gather(gb):
        pltpu.make_async_copy(
            wpad_hbm.at[idx_v.at[0, 0]], rows_v.at[gb], sem_g[gb]
        ).wait()

    rowvecs = [iota16 + j0 * 16 for j0 in range(8)]

    def transpose_one(gb):
        def dbody(d, carry):
            col = jnp.full((16,), d, jnp.int32)
            for j0 in range(8):
                vec = plsc.load_gather(rows_v.at[gb], [rowvecs[j0], col])
                tile_v[gb, d, pl.ds(j0 * 16, 16)] = vec
            return carry
        lax.fori_loop(0, D, dbody, 0, unroll=4)

    def start_out(u, r, gb):
        tt = u // 128
        bc = lax.rem(u, 128)
        pltpu.make_async_copy(
            tile_v.at[gb],
            out_hbm.at[tt * 8 + r, :, pl.ds(bc * 128, 128)],
            sem_o[gb],
        ).start()

    def wait_out(gb):
        pltpu.make_async_copy(
            tile_v.at[gb], out_hbm.at[0, :, pl.ds(0, 128)], sem_o[gb]
        ).wait()

    start_idx(u0, 0)
    start_idx(u0 + 1, 1)
    wait_idx(0)
    start_gather(0, 0, 0)
    start_out(u0, 0, 0)
    start_out(u0, 1, 1)

    last_p = NU // 2 - 1

    def pair(p, carry):
        ua = u0 + 2 * p
        for up in range(2):
            u = ua + up
            for r in range(8):
                gb = r % 2
                wait_gather(gb)
                if r < 7:
                    start_gather(up, r + 1, 1 - gb)
                elif up == 0:
                    wait_idx(1)
                    start_gather(1, 0, 1 - gb)
                else:
                    @pl.when(p < last_p)
                    def _():
                        wait_idx(0)
                        start_gather(0, 0, 1 - gb)
                wait_out(gb)
                transpose_one(gb)
                start_out(u, r, gb)

            @pl.when(p < last_p)
            def _():
                start_idx(u + 2, up)
        return carry

    lax.fori_loop(0, NU // 2, pair, 0, unroll=False)
    for gb in range(2):
        wait_out(gb)


def kernel(token_ids, W):
    wt = jnp.swapaxes(W, 0, 1)
    ids = jnp.swapaxes(token_ids, 0, 1).astype(jnp.int32)
    mesh = plsc.VectorSubcoreMesh(core_axis_name="c", subcore_axis_name="s")
    cp = pltpu.CompilerParams(use_tc_tiling_on_sc=True, needs_layout_passes=False)

    wpad = pl.kernel(
        _wfmt_body,
        mesh=mesh,
        compiler_params=cp,
        out_type=jax.ShapeDtypeStruct((V, 128), jnp.float32),
        scratch_types=[
            pltpu.VMEM((2, D, VBLK), jnp.float32),
            pltpu.VMEM((2, VBLK, 128), jnp.float32),
            pltpu.VMEM((D, TAIL), jnp.float32),
            pltpu.VMEM((TAIL, 128), jnp.float32),
            pltpu.SemaphoreType.DMA,
            pltpu.SemaphoreType.DMA,
            pltpu.SemaphoreType.DMA,
            pltpu.SemaphoreType.DMA,
        ],
    )(wt)

    out = pl.kernel(
        _gather_body,
        mesh=mesh,
        compiler_params=cp,
        out_type=jax.ShapeDtypeStruct((T, D, B), jnp.float32),
        scratch_types=[
            pltpu.VMEM((2, 8, 128), jnp.int32),
            pltpu.VMEM((2, 128, 128), jnp.float32),
            pltpu.VMEM((2, D, 128), jnp.float32),
            pltpu.SemaphoreType.DMA,
            pltpu.SemaphoreType.DMA,
            pltpu.SemaphoreType.DMA,
            pltpu.SemaphoreType.DMA,
            pltpu.SemaphoreType.DMA,
            pltpu.SemaphoreType.DMA,
            pltpu.SemaphoreType.DMA,
            pltpu.SemaphoreType.DMA,
        ],
    )(ids, wpad)

    return jnp.transpose(out, (2, 0, 1))


# speedup vs baseline: 1.2136x; 1.2136x over previous
import jax
import jax.numpy as jnp
from jax import lax
from jax.experimental import pallas as pl
from jax.experimental.pallas import tpu as pltpu
from jax.experimental.pallas import tpu_sc as plsc

EMBEDDING_DIM = 32
NUM_CORES = 2
NUM_SUBCORES = 16
NUM_WORKERS = NUM_CORES * NUM_SUBCORES

V = 1_000_000
D = EMBEDDING_DIM
VBLK = 128
NFULL = V // VBLK
TAIL = V - NFULL * VBLK
PACKROWS = V * D // 128

CHUNK = 1600
NBUF = 2


def _wid():
    return lax.axis_index("s") * NUM_CORES + lax.axis_index("c")


def _wpack_body(wt_hbm, wpack_hbm, band_v, tr_v, band_t, tr_t,
                sem_i0, sem_i1, sem_o0, sem_o1):
    wid = _wid()
    nblk = (NFULL - wid + NUM_WORKERS - 1) // NUM_WORKERS
    iota16 = lax.iota(jnp.int32, 16)
    base32 = [(iota16 + j0 * 16) * D for j0 in range(8)]
    sem_i = (sem_i0, sem_i1)
    sem_o = (sem_o0, sem_o1)

    def start_in(vb, bb):
        pltpu.make_async_copy(
            wt_hbm.at[:, pl.ds(vb * VBLK, VBLK)], band_v.at[bb], sem_i[bb]
        ).start()

    def wait_in(bb):
        pltpu.make_async_copy(
            wt_hbm.at[:, pl.ds(0, VBLK)], band_v.at[bb], sem_i[bb]
        ).wait()

    def start_out(vb, bb):
        pltpu.make_async_copy(
            tr_v.at[bb], wpack_hbm.at[pl.ds(vb * (VBLK * D // 128), VBLK * D // 128)],
            sem_o[bb],
        ).start()

    def wait_out(bb):
        pltpu.make_async_copy(
            tr_v.at[bb], wpack_hbm.at[pl.ds(0, VBLK * D // 128)], sem_o[bb]
        ).wait()

    def transpose_block(bb):
        def dbody(d, carry):
            for j0 in range(8):
                vec = band_v[bb, d, pl.ds(j0 * 16, 16)]
                flat = base32[j0] + d
                plsc.store_scatter(
                    tr_v.at[bb],
                    [lax.shift_right_logical(flat, 7),
                     lax.bitwise_and(flat, 127)],
                    vec)
            return carry
        lax.fori_loop(0, D, dbody, 0, unroll=2)

    start_in(wid, 0)
    start_in(wid + NUM_WORKERS, 1)
    for bb in range(2):
        wait_in(bb)
        transpose_block(bb)
        start_out(wid + bb * NUM_WORKERS, bb)
        start_in(wid + (bb + 2) * NUM_WORKERS, bb)

    def group(g, carry):
        for bb in range(2):
            k = 2 + g * 2 + bb
            vb = wid + k * NUM_WORKERS

            @pl.when(k < nblk)
            def _():
                wait_in(bb)
                wait_out(bb)
                transpose_block(bb)
                start_out(vb, bb)

            @pl.when(k + 2 < nblk)
            def _():
                pltpu.make_async_copy(
                    wt_hbm.at[:, pl.ds((vb + 2 * NUM_WORKERS) * VBLK, VBLK)],
                    band_v.at[bb], sem_i[bb],
                ).start()
        return carry

    ngroups = (245 - 2 + 1) // 2
    lax.fori_loop(0, ngroups, group, 0, unroll=False)
    for bb in range(2):
        wait_out(bb)

    @pl.when(wid == NUM_WORKERS - 1)
    def _():
        pltpu.sync_copy(wt_hbm.at[:, pl.ds(NFULL * VBLK, TAIL)], band_t)

        def dbody(d, carry):
            for j0 in range(TAIL // 16):
                vec = band_t[d, pl.ds(j0 * 16, 16)]
                flat = base32[j0] + d
                plsc.store_scatter(
                    tr_t,
                    [lax.shift_right_logical(flat, 7),
                     lax.bitwise_and(flat, 127)],
                    vec)
            return carry
        lax.fori_loop(0, D, dbody, 0, unroll=2)
        pltpu.sync_copy(
            tr_t, wpack_hbm.at[pl.ds(NFULL * (VBLK * D // 128), TAIL * D // 128)])


def _emb_body(idx_hbm, table_hbm, out_hbm,
              idx_v, rows_v, sem_i0, sem_i1, sem_g0, sem_g1, sem_s0, sem_s1):
    wid = _wid()
    n_per_w = idx_hbm.shape[0] // NUM_WORKERS
    nchunks = n_per_w // CHUNK
    base = wid * n_per_w

    sem_i = (sem_i0, sem_i1)
    sem_g = (sem_g0, sem_g1)
    sem_s = (sem_s0, sem_s1)

    def start_idx(g, b):
        pltpu.make_async_copy(
            idx_hbm.at[pl.ds(base + g * CHUNK, CHUNK)], idx_v.at[b], sem_i[b]
        ).start()

    def wait_idx(b):
        pltpu.make_async_copy(
            idx_hbm.at[pl.ds(base, CHUNK)], idx_v.at[b], sem_i[b]
        ).wait()

    def start_gather(b):
        pltpu.make_async_copy(
            table_hbm.at[idx_v.at[b]], rows_v.at[b], sem_g[b]
        ).start()

    def wait_gather(b):
        pltpu.make_async_copy(
            table_hbm.at[idx_v.at[b]], rows_v.at[b], sem_g[b]
        ).wait()

    def start_store(g, b):
        pltpu.make_async_copy(
            rows_v.at[b], out_hbm.at[pl.ds(base + g * CHUNK, CHUNK)], sem_s[b]
        ).start()

    def wait_store(b):
        pltpu.make_async_copy(
            rows_v.at[b], out_hbm.at[pl.ds(base, CHUNK)], sem_s[b]
        ).wait()

    start_idx(0, 0)
    start_idx(1, 1)
    for b in range(NBUF):
        wait_idx(b)
        start_gather(b)
        wait_gather(b)
        start_store(b, b)
        start_idx(b + NBUF, b)

    def group_body(gr, carry):
        for b in range(NBUF):
            g = NBUF + gr * NBUF + b
            wait_idx(b)
            wait_store(b)
            start_gather(b)
            wait_gather(b)
            start_store(g, b)
            start_idx(g + NBUF, b)
        return carry

    ngroups = (nchunks - 2 * NBUF) // NBUF
    lax.fori_loop(0, ngroups, group_body, 0, unroll=False)

    for b in range(NBUF):
        g = nchunks - NBUF + b
        wait_idx(b)
        wait_store(b)
        start_gather(b)
        wait_gather(b)
        start_store(g, b)
    for b in range(NBUF):
        wait_store(b)


def kernel(token_ids, W):
    Bsz, H = token_ids.shape
    flat = token_ids.reshape(-1).astype(jnp.int32)
    n = flat.shape[0]
    wt = jnp.swapaxes(W, 0, 1)
    mesh = plsc.VectorSubcoreMesh(core_axis_name="c", subcore_axis_name="s")

    wpack = pl.kernel(
        _wpack_body,
        mesh=mesh,
        compiler_params=pltpu.CompilerParams(
            use_tc_tiling_on_sc=True, needs_layout_passes=False),
        out_type=jax.ShapeDtypeStruct((PACKROWS, 128), jnp.float32),
        scratch_types=[
            pltpu.VMEM((2, D, VBLK), jnp.float32),
            pltpu.VMEM((2, VBLK * D // 128, 128), jnp.float32),
            pltpu.VMEM((D, TAIL), jnp.float32),
            pltpu.VMEM((TAIL * D // 128, 128), jnp.float32),
            pltpu.SemaphoreType.DMA,
            pltpu.SemaphoreType.DMA,
            pltpu.SemaphoreType.DMA,
            pltpu.SemaphoreType.DMA,
        ],
    )(wt)
    table = wpack.reshape(V, D)

    out = pl.kernel(
        _emb_body,
        mesh=mesh,
        compiler_params=pltpu.CompilerParams(use_tc_tiling_on_sc=False),
        out_type=jax.ShapeDtypeStruct((n, EMBEDDING_DIM), jnp.float32),
        scratch_types=[
            pltpu.VMEM((NBUF, CHUNK), jnp.int32),
            pltpu.VMEM((NBUF, CHUNK, EMBEDDING_DIM), jnp.float32),
            pltpu.SemaphoreType.DMA,
            pltpu.SemaphoreType.DMA,
            pltpu.SemaphoreType.DMA,
            pltpu.SemaphoreType.DMA,
            pltpu.SemaphoreType.DMA,
            pltpu.SemaphoreType.DMA,
        ],
    )(flat, table)
    return out.reshape(Bsz, H, EMBEDDING_DIM)
